# baseline (speedup 1.0000x reference)
import jax
import jax.numpy as jnp
from jax.experimental import pallas as pl
from jax.experimental.pallas import tpu as pltpu

_N = 5000
_D = 12544
_H = 1024
_O = 16
_TN = 64


def _boxhead_body(fv_ref, w1_ref, b1_ref, w2_ref, b2_ref, wh_ref, bh_ref,
                  out_ref):
    x = jnp.dot(fv_ref[...], w1_ref[...], preferred_element_type=jnp.float32)
    x = jnp.maximum(x + b1_ref[...], 0.0)
    x = jnp.dot(x, w2_ref[...], preferred_element_type=jnp.float32)
    x = jnp.maximum(x + b2_ref[...], 0.0)
    out_ref[...] = (
        jnp.dot(x, wh_ref[...], preferred_element_type=jnp.float32)
        + bh_ref[...]
    )


def kernel(feature_vectors, W1, b1, W2, b2, Wc, bc, Wr, br):
    Wh = jnp.concatenate([Wc, Wr], axis=1)
    bh = jnp.concatenate([bc, br])[None, :]
    out = pl.pallas_call(
        _boxhead_body,
        grid=(pl.cdiv(_N, _TN),),
        in_specs=[
            pl.BlockSpec((_TN, _D), lambda i: (i, 0)),
            pl.BlockSpec((_D, _H), lambda i: (0, 0)),
            pl.BlockSpec((1, _H), lambda i: (0, 0)),
            pl.BlockSpec((_H, _H), lambda i: (0, 0)),
            pl.BlockSpec((1, _H), lambda i: (0, 0)),
            pl.BlockSpec((_H, _O), lambda i: (0, 0)),
            pl.BlockSpec((1, _O), lambda i: (0, 0)),
        ],
        out_specs=pl.BlockSpec((_TN, _O), lambda i: (i, 0)),
        out_shape=jax.ShapeDtypeStruct((_N, _O), jnp.float32),
        compiler_params=pltpu.CompilerParams(vmem_limit_bytes=100 * 1024 * 1024),
    )(feature_vectors, W1, b1[None, :], W2, b2[None, :], Wh, bh)
    return out[:, :4], out[:, 4:]


# speedup vs baseline: 2.0989x; 2.0989x over previous
import jax
import jax.numpy as jnp
from jax.experimental import pallas as pl
from jax.experimental.pallas import tpu as pltpu

_N = 5000
_D = 12544
_H = 1024
_O = 16
_TN = 256


def _boxhead_body(fv_ref, w1_ref, b1_ref, w2_ref, b2_ref, wh_ref, bh_ref,
                  out_ref):
    fv = fv_ref[...].astype(jnp.bfloat16)
    x = jnp.dot(fv, w1_ref[...], preferred_element_type=jnp.float32)
    x = jnp.maximum(x + b1_ref[...], 0.0).astype(jnp.bfloat16)
    x = jnp.dot(x, w2_ref[...], preferred_element_type=jnp.float32)
    x = jnp.maximum(x + b2_ref[...], 0.0).astype(jnp.bfloat16)
    out_ref[...] = (
        jnp.dot(x, wh_ref[...], preferred_element_type=jnp.float32)
        + bh_ref[...]
    )


def kernel(feature_vectors, W1, b1, W2, b2, Wc, bc, Wr, br):
    Wh = jnp.concatenate([Wc, Wr], axis=1).astype(jnp.bfloat16)
    bh = jnp.concatenate([bc, br])[None, :]
    out = pl.pallas_call(
        _boxhead_body,
        grid=(pl.cdiv(_N, _TN),),
        in_specs=[
            pl.BlockSpec((_TN, _D), lambda i: (i, 0)),
            pl.BlockSpec((_D, _H), lambda i: (0, 0)),
            pl.BlockSpec((1, _H), lambda i: (0, 0)),
            pl.BlockSpec((_H, _H), lambda i: (0, 0)),
            pl.BlockSpec((1, _H), lambda i: (0, 0)),
            pl.BlockSpec((_H, _O), lambda i: (0, 0)),
            pl.BlockSpec((1, _O), lambda i: (0, 0)),
        ],
        out_specs=pl.BlockSpec((_TN, _O), lambda i: (i, 0)),
        out_shape=jax.ShapeDtypeStruct((_N, _O), jnp.float32),
        compiler_params=pltpu.CompilerParams(vmem_limit_bytes=100 * 1024 * 1024),
    )(feature_vectors, W1.astype(jnp.bfloat16), b1[None, :],
      W2.astype(jnp.bfloat16), b2[None, :], Wh, bh)
    return out[:, :4], out[:, 4:]
